# Initial kernel scaffold; baseline (speedup 1.0000x reference)
import jax
import jax.numpy as jnp
from jax.experimental import pallas as pl


def kernel(meeting_utterance_enc_hidden_states, adj_coos, edge_types, basis1, att1, gate1, root1, bias1, basis2, att2, gate2, root2, bias2):
    raise NotImplementedError("write your pallas kernel here")


# speedup vs baseline: 16.2350x; 16.2350x over previous
import functools

import jax
import jax.numpy as jnp
from jax import lax
from jax.experimental import pallas as pl
from jax.experimental.pallas import tpu as pltpu
from jax.experimental.pallas import tpu_sc as plsc

N = 10000
N2 = 10240
D = 128
E = 320000
R = 2
NB = 30

NS = 16
CHUNK = 128
NCHUNK = 160
E2 = NS * NCHUNK * CHUNK
ROWS_PT = N2 // NS
STAGE = 128
NSTAGE = ROWS_PT // STAGE

BN = 1024
NBLK = N2 // BN


def _w_body(att_ref, basis_ref, w_ref):
    w_ref[...] = jax.lax.dot_general(
        att_ref[...], basis_ref[...], (((1,), (0,)), ((), ())),
        preferred_element_type=jnp.float32, precision=lax.Precision.HIGHEST)


_w_call = pl.pallas_call(
    _w_body,
    out_shape=jax.ShapeDtypeStruct((R, D * D), jnp.float32),
)


def _prep_body(xin_ref, w_ref, gwt_ref, root_ref, bias_ref, y_ref, init_ref,
               *, relu):
    x = jax.nn.relu(xin_ref[...]) if relu else xin_ref[...]
    dot = functools.partial(
        jax.lax.dot_general, dimension_numbers=(((1,), (0,)), ((), ())),
        preferred_element_type=jnp.float32, precision=lax.Precision.HIGHEST)
    s = jax.nn.sigmoid(dot(x, gwt_ref[...]))
    for r in range(R):
        y_ref[r] = s[:, r:r + 1] * dot(x, w_ref[r])
    init_ref[...] = dot(x, root_ref[...]) + bias_ref[...]


def _make_prep(relu):
    return pl.pallas_call(
        functools.partial(_prep_body, relu=relu),
        grid=(NBLK,),
        in_specs=[
            pl.BlockSpec((BN, D), lambda n: (n, 0)),
            pl.BlockSpec((R, D, D), lambda n: (0, 0, 0)),
            pl.BlockSpec((D, R), lambda n: (0, 0)),
            pl.BlockSpec((D, D), lambda n: (0, 0)),
            pl.BlockSpec((1, D), lambda n: (0, 0)),
        ],
        out_specs=[
            pl.BlockSpec((R, BN, D), lambda n: (0, n, 0)),
            pl.BlockSpec((BN, D), lambda n: (n, 0)),
        ],
        out_shape=[
            jax.ShapeDtypeStruct((R, N2, D), jnp.float32),
            jax.ShapeDtypeStruct((N2, D), jnp.float32),
        ],
    )


_prep1 = _make_prep(relu=False)
_prep2 = _make_prep(relu=True)


def _gidx_body(j_ref, et_ref, g_ref):
    g_ref[...] = et_ref[...] * N2 + j_ref[...]


_gidx_call = pl.pallas_call(
    _gidx_body,
    out_shape=jax.ShapeDtypeStruct((E // 128, 128), jnp.int32),
)


_sc_mesh = plsc.VectorSubcoreMesh(core_axis_name="c", subcore_axis_name="s")


@functools.partial(
    pl.kernel,
    out_type=jax.ShapeDtypeStruct((N2, D), jnp.float32),
    mesh=_sc_mesh,
    scratch_types=[
        pltpu.VMEM_SHARED((N2, D), jnp.float32),
        pltpu.VMEM((2, CHUNK), jnp.int32),
        pltpu.VMEM((2, CHUNK), jnp.int32),
        pltpu.VMEM((CHUNK, D), jnp.float32),
        pltpu.VMEM((CHUNK, D), jnp.float32),
        pltpu.SemaphoreType.DMA,
        pltpu.SemaphoreType.DMA,
    ],
)
def _sc_edges(y_hbm, idx_hbm, init_hbm, out_hbm,
              acc, ibufA, ibufB, rowsA, rowsB, semA, semB):
    cid = lax.axis_index("c")
    sid = lax.axis_index("s")

    @pl.when(cid == 0)
    def _():
        base = sid * NCHUNK

        for k in range(NSTAGE):
            row0 = sid * ROWS_PT + k * STAGE
            pltpu.sync_copy(init_hbm.at[pl.ds(row0, STAGE)], rowsA)
            pltpu.sync_copy(rowsA, acc.at[pl.ds(row0, STAGE)])
        plsc.subcore_barrier()

        pltpu.sync_copy(idx_hbm.at[base], ibufA)
        pltpu.async_copy(y_hbm.at[ibufA.at[0]], rowsA, semA)

        def pair(kk, carry):
            a = base + 2 * kk
            pltpu.sync_copy(idx_hbm.at[a + 1], ibufB)
            pltpu.async_copy(y_hbm.at[ibufB.at[0]], rowsB, semB)
            pltpu.make_async_copy(y_hbm.at[ibufA.at[0]], rowsA, semA).wait()
            pltpu.sync_copy(rowsA, acc.at[ibufA.at[1]], add=True)

            @pl.when(kk < NCHUNK // 2 - 1)
            def _():
                pltpu.sync_copy(idx_hbm.at[a + 2], ibufA)
                pltpu.async_copy(y_hbm.at[ibufA.at[0]], rowsA, semA)

            pltpu.make_async_copy(y_hbm.at[ibufB.at[0]], rowsB, semB).wait()
            pltpu.sync_copy(rowsB, acc.at[ibufB.at[1]], add=True)
            return carry

        lax.fori_loop(0, NCHUNK // 2, pair, 0, unroll=False)
        plsc.subcore_barrier()

        for k in range(NSTAGE):
            row0 = sid * ROWS_PT + k * STAGE
            pltpu.sync_copy(acc.at[pl.ds(row0, STAGE)], rowsA)
            pltpu.sync_copy(rowsA, out_hbm.at[pl.ds(row0, STAGE)])


def _layer(xin, idx3, w, gwt, root, bias, relu):
    prep = _prep2 if relu else _prep1
    y, init = prep(xin, w, gwt, root, bias)
    return _sc_edges(y.reshape(R * N2, D), idx3, init)


def kernel(meeting_utterance_enc_hidden_states, adj_coos, edge_types,
           basis1, att1, gate1, root1, bias1,
           basis2, att2, gate2, root2, bias2):
    x = jnp.pad(meeting_utterance_enc_hidden_states,
                ((0, 0), (0, N2 - N), (0, 0)))[0]
    i_idx = adj_coos[0, 0]
    j_idx = adj_coos[0, 1]
    et = edge_types[0]

    gidx = _gidx_call(j_idx.reshape(E // 128, 128), et.reshape(E // 128, 128))
    gidx3 = jnp.pad(gidx.reshape(E), (0, E2 - E)).reshape(NS * NCHUNK, 1, CHUNK)
    didx3 = jnp.pad(i_idx, (0, E2 - E),
                    constant_values=N).reshape(NS * NCHUNK, 1, CHUNK)
    idx3 = jnp.concatenate([gidx3, didx3], axis=1)

    w1 = _w_call(att1, basis1.reshape(NB, D * D)).reshape(R, D, D)
    w2 = _w_call(att2, basis2.reshape(NB, D * D)).reshape(R, D, D)
    gwt1 = gate1[:, :, 0].T
    gwt2 = gate2[:, :, 0].T
    bias1_2d = bias1.reshape(1, D)
    bias2_2d = bias2.reshape(1, D)

    p1 = _layer(x, idx3, w1, gwt1, root1, bias1_2d, relu=False)
    p2 = _layer(p1, idx3, w2, gwt2, root2, bias2_2d, relu=True)
    return p2[:N]
